# Initial kernel scaffold; baseline (speedup 1.0000x reference)
import jax
import jax.numpy as jnp
from jax.experimental import pallas as pl


def kernel(expert_assignments):
    raise NotImplementedError("write your pallas kernel here")


# speedup vs baseline: 79.9399x; 79.9399x over previous
import functools

import jax
import jax.numpy as jnp
from jax import lax
from jax.experimental import pallas as pl
from jax.experimental.pallas import tpu as pltpu
from jax.experimental.pallas import tpu_sc as plsc

NUM_TOKENS = 32768
NUM_EXPERTS = 64
CAPACITY = 512.0
L = 16
NS = 16
CHUNK = NUM_TOKENS // NS
GROUPS = NUM_EXPERTS // L

_mesh = plsc.VectorSubcoreMesh(
    core_axis_name="c", subcore_axis_name="s", num_cores=1)


def _capacity_loss_body(assign_hbm, out_hbm, vals_v, hist2d_v, hist_v,
                        shared_v, all_v, out_v):
    sid = lax.axis_index("s").astype(jnp.int32)

    pltpu.sync_copy(assign_hbm.at[pl.ds(sid * jnp.int32(CHUNK), CHUNK)], vals_v)

    zeros = jnp.zeros((L,), jnp.int32)
    for l in range(L):
        for g in range(GROUPS):
            hist2d_v[l, pl.ds(g * L, L)] = zeros

    lane = lax.iota(jnp.int32, L)
    ones = jnp.ones((L,), jnp.int32)

    def scatter_step(i, carry):
        v = vals_v[pl.ds(i * jnp.int32(L), L)]
        plsc.addupdate_scatter(hist2d_v, [lane, v], ones)
        return carry

    lax.fori_loop(jnp.int32(0), jnp.int32(CHUNK // L), scatter_step,
                  jnp.int32(0))

    for g in range(GROUPS):
        acc = hist2d_v[0, pl.ds(g * L, L)]
        for l in range(1, L):
            acc = acc + hist2d_v[l, pl.ds(g * L, L)]
        hist_v[pl.ds(g * L, L)] = acc

    pltpu.sync_copy(hist_v, shared_v.at[sid])
    plsc.subcore_barrier()

    @pl.when(sid == 0)
    def _finalize():
        pltpu.sync_copy(shared_v, all_v)
        total = jnp.zeros((), jnp.float32)
        n_exp = jnp.zeros((), jnp.int32)
        for g in range(GROUPS):
            acc = all_v[0, pl.ds(g * L, L)]
            for t in range(1, NS):
                acc = acc + all_v[t, pl.ds(g * L, L)]
            cand = jnp.where(acc > 0, lane + jnp.int32(g * L + 1), jnp.int32(0))
            n_exp = jnp.maximum(n_exp, jnp.max(cand))
            over = jnp.maximum(acc.astype(jnp.float32) - CAPACITY, 0.0)
            total = total + jnp.sum(over * over)
        loss = jnp.full((L,), total) / jnp.full((L,), n_exp).astype(jnp.float32)
        out_v[...] = loss
        pltpu.sync_copy(out_v, out_hbm)


_SCRATCH_TYPES = [
    pltpu.VMEM((CHUNK,), jnp.int32),
    pltpu.VMEM((L, NUM_EXPERTS), jnp.int32),
    pltpu.VMEM((NUM_EXPERTS,), jnp.int32),
    pltpu.VMEM_SHARED((NS, NUM_EXPERTS), jnp.int32),
    pltpu.VMEM((NS, NUM_EXPERTS), jnp.int32),
    pltpu.VMEM((L,), jnp.float32),
]

_capacity_loss = pl.kernel(
    _capacity_loss_body,
    mesh=_mesh,
    out_type=jax.ShapeDtypeStruct((L,), jnp.float32),
    scratch_types=_SCRATCH_TYPES,
    compiler_params=pltpu.CompilerParams(needs_layout_passes=False),
)


def kernel(expert_assignments):
    a = expert_assignments.astype(jnp.int32)
    return _capacity_loss(a)[0]
